# baseline (speedup 1.0000x reference)
import functools

import jax
import jax.numpy as jnp
from jax import lax
from jax.experimental import pallas as pl
from jax.experimental.pallas import tpu as pltpu
from jax.experimental.pallas import tpu_sc as plsc

N = 10000
E = 320000
D_IN = 128
HID = 512
NPAD = 10240
NC = 2
NS = 16
CHUNK = 128
K = 80
KM = 78
TAIL = 16
EMAIN = NC * NS * KM * CHUNK
ROWB = 512
NBLK = NPAD // ROWB
TILE_ROWS = NPAD // NS

_mesh = plsc.VectorSubcoreMesh(core_axis_name="c", subcore_axis_name="s")
_sc_params = pltpu.CompilerParams(needs_layout_passes=False)


def _fill_idx(main_hbm, tail_hbm, cid, sid, buf):
    pltpu.sync_copy(main_hbm.at[cid, sid], buf.at[pl.ds(0, KM)])
    pltpu.sync_copy(tail_hbm.at[cid, sid], buf.at[KM, pl.ds(0, TAIL)])
    lanes = lax.iota(jnp.int32, 16)
    for i in range(7):
        buf[KM, pl.ds(TAIL + 16 * i, 16)] = N + 16 * i + lanes
    for i in range(8):
        buf[KM + 1, pl.ds(16 * i, 16)] = N + 112 + 16 * i + lanes


@functools.partial(
    pl.kernel,
    out_type=jax.ShapeDtypeStruct((NC * NS, NPAD), jnp.float32),
    mesh=_mesh,
    compiler_params=_sc_params,
    scratch_types=[
        pltpu.VMEM((NPAD,), jnp.float32),
        pltpu.VMEM((K, CHUNK), jnp.int32),
    ],
)
def _sc_degree(dstm_hbm, dstt_hbm, out_hbm, hist, dall):
    cid = lax.axis_index("c")
    sid = lax.axis_index("s")
    wid = cid * NS + sid
    zeros16 = jnp.zeros((16,), jnp.float32)
    ones16 = jnp.ones((16,), jnp.float32)
    _fill_idx(dstm_hbm, dstt_hbm, cid, sid, dall)

    def zero_body(i, _):
        hist[pl.ds(i * 16, 16)] = zeros16
        return 0

    lax.fori_loop(0, NPAD // 16, zero_body, 0)

    def chunk_body(j, _):
        for i in range(CHUNK // 16):
            idx = dall[j, pl.ds(i * 16, 16)]
            plsc.addupdate_scatter(hist, [idx], ones16)
        return 0

    lax.fori_loop(0, K, chunk_body, 0)
    pltpu.sync_copy(hist, out_hbm.at[wid])


@functools.partial(
    pl.kernel,
    out_type=jax.ShapeDtypeStruct((NC, NPAD, D_IN), jnp.float32),
    mesh=_mesh,
    compiler_params=_sc_params,
    scratch_types=[
        pltpu.VMEM_SHARED((NPAD, D_IN), jnp.float32),
        pltpu.VMEM((CHUNK, D_IN), jnp.float32),
        pltpu.VMEM((CHUNK, D_IN), jnp.float32),
        pltpu.VMEM((4, CHUNK), jnp.int32),
        pltpu.VMEM((K, CHUNK), jnp.int32),
        pltpu.VMEM((2, CHUNK), jnp.int32),
        pltpu.SemaphoreType.DMA,
        pltpu.SemaphoreType.DMA,
        pltpu.SemaphoreType.DMA,
        pltpu.SemaphoreType.DMA,
        pltpu.SemaphoreType.DMA,
        pltpu.SemaphoreType.DMA,
        pltpu.SemaphoreType.DMA,
        pltpu.SemaphoreType.DMA,
    ],
)
def _sc_seg128(srcm_hbm, srct_hbm, dstm_hbm, dstt_hbm, a_hbm, out_hbm,
               acc, rows0, rows1, sring, dall, stail,
               gs0, gs1, ss0, ss1, is0, is1, is2, is3):
    cid = lax.axis_index("c")
    sid = lax.axis_index("s")
    zeros16 = jnp.zeros((16,), jnp.float32)
    _fill_idx(dstm_hbm, dstt_hbm, cid, sid, dall)
    pltpu.sync_copy(srct_hbm.at[cid, sid], stail.at[0, pl.ds(0, TAIL)])
    lanes = lax.iota(jnp.int32, 16)
    for i in range(7):
        stail[0, pl.ds(TAIL + 16 * i, 16)] = N + 16 * i + lanes
    for i in range(8):
        stail[1, pl.ds(16 * i, 16)] = N + 112 + 16 * i + lanes

    def zero_rows(i, _):
        r = i // (D_IN // 16)
        col = i % (D_IN // 16)
        rows0[r, pl.ds(col * 16, 16)] = zeros16
        return 0

    lax.fori_loop(0, CHUNK * (D_IN // 16), zero_rows, 0)
    base = sid * TILE_ROWS
    for k in range(TILE_ROWS // CHUNK):
        pltpu.sync_copy(rows0, acc.at[pl.ds(base + k * CHUNK, CHUNK)])
    plsc.subcore_barrier()

    isems = [is0, is1, is2, is3]

    def idx_start(jj, slot):
        pltpu.async_copy(srcm_hbm.at[cid, sid, jj], sring.at[slot],
                         isems[slot])

    def idx_wait(jj, slot):
        pltpu.make_async_copy(srcm_hbm.at[cid, sid, jj], sring.at[slot],
                              isems[slot]).wait()

    def gat_start(slot, rows, gsem):
        pltpu.async_copy(a_hbm.at[sring.at[slot]], rows, gsem)

    def gat_wait(slot, rows, gsem):
        pltpu.make_async_copy(a_hbm.at[sring.at[slot]], rows, gsem).wait()

    def sca_start(jj, rows, ssem):
        pltpu.async_copy(rows, acc.at[dall.at[jj]], ssem, add=True)

    def sca_wait(jj, rows, ssem):
        pltpu.make_async_copy(rows, acc.at[dall.at[jj]], ssem).wait()

    pltpu.sync_copy(srcm_hbm.at[cid, sid, 0], sring.at[0])
    pltpu.sync_copy(srcm_hbm.at[cid, sid, 1], sring.at[1])
    gat_start(0, rows0, gs0)
    gat_start(1, rows1, gs1)
    idx_start(2, 2)
    idx_start(3, 3)

    def pipe_body(it, _):
        j = 4 * it
        gat_wait(0, rows0, gs0)
        idx_start(j + 4, 0)
        sca_start(j, rows0, ss0)
        gat_wait(1, rows1, gs1)
        idx_start(j + 5, 1)
        sca_start(j + 1, rows1, ss1)
        sca_wait(j, rows0, ss0)
        idx_wait(j + 2, 2)
        gat_start(2, rows0, gs0)
        sca_wait(j + 1, rows1, ss1)
        idx_wait(j + 3, 3)
        gat_start(3, rows1, gs1)
        gat_wait(2, rows0, gs0)
        idx_start(j + 6, 2)
        sca_start(j + 2, rows0, ss0)
        gat_wait(3, rows1, gs1)
        idx_start(j + 7, 3)
        sca_start(j + 3, rows1, ss1)
        sca_wait(j + 2, rows0, ss0)
        idx_wait(j + 4, 0)
        gat_start(0, rows0, gs0)
        sca_wait(j + 3, rows1, ss1)
        idx_wait(j + 5, 1)
        gat_start(1, rows1, gs1)
        return 0

    lax.fori_loop(0, (K - 4) // 4 - 1, pipe_body, 0)

    j = K - 8
    gat_wait(0, rows0, gs0)
    idx_start(j + 4, 0)
    sca_start(j, rows0, ss0)
    gat_wait(1, rows1, gs1)
    idx_start(j + 5, 1)
    sca_start(j + 1, rows1, ss1)
    sca_wait(j, rows0, ss0)
    idx_wait(j + 2, 2)
    gat_start(2, rows0, gs0)
    sca_wait(j + 1, rows1, ss1)
    idx_wait(j + 3, 3)
    gat_start(3, rows1, gs1)
    gat_wait(2, rows0, gs0)
    sca_start(j + 2, rows0, ss0)
    gat_wait(3, rows1, gs1)
    sca_start(j + 3, rows1, ss1)
    sca_wait(j + 2, rows0, ss0)
    idx_wait(j + 4, 0)
    gat_start(0, rows0, gs0)
    sca_wait(j + 3, rows1, ss1)
    idx_wait(j + 5, 1)
    gat_start(1, rows1, gs1)

    jl = K - 4
    gat_wait(0, rows0, gs0)
    sca_start(jl, rows0, ss0)
    gat_wait(1, rows1, gs1)
    sca_start(jl + 1, rows1, ss1)
    sca_wait(jl, rows0, ss0)
    pltpu.async_copy(a_hbm.at[stail.at[0]], rows0, gs0)
    sca_wait(jl + 1, rows1, ss1)
    pltpu.async_copy(a_hbm.at[stail.at[1]], rows1, gs1)
    pltpu.make_async_copy(a_hbm.at[stail.at[0]], rows0, gs0).wait()
    sca_start(jl + 2, rows0, ss0)
    pltpu.make_async_copy(a_hbm.at[stail.at[1]], rows1, gs1).wait()
    sca_start(jl + 3, rows1, ss1)
    sca_wait(jl + 2, rows0, ss0)
    sca_wait(jl + 3, rows1, ss1)

    plsc.subcore_barrier()
    pltpu.sync_copy(acc.at[pl.ds(base, TILE_ROWS)],
                    out_hbm.at[cid, pl.ds(base, TILE_ROWS)])


@functools.partial(
    pl.kernel,
    out_type=jax.ShapeDtypeStruct((NC * NS, NPAD), jnp.float32),
    mesh=_mesh,
    compiler_params=_sc_params,
    scratch_types=[
        pltpu.VMEM((NPAD,), jnp.float32),
        pltpu.VMEM((NPAD,), jnp.float32),
        pltpu.VMEM((K, CHUNK), jnp.int32),
        pltpu.VMEM((K, CHUNK), jnp.int32),
    ],
)
def _sc_segscalar(srcm_hbm, srct_hbm, dstm_hbm, dstt_hbm, c_hbm, out_hbm,
                  cloc, tloc, sall, dall):
    cid = lax.axis_index("c")
    sid = lax.axis_index("s")
    wid = cid * NS + sid
    zeros16 = jnp.zeros((16,), jnp.float32)
    _fill_idx(srcm_hbm, srct_hbm, cid, sid, sall)
    _fill_idx(dstm_hbm, dstt_hbm, cid, sid, dall)
    pltpu.sync_copy(c_hbm, cloc)

    def zero_body(i, _):
        tloc[pl.ds(i * 16, 16)] = zeros16
        return 0

    lax.fori_loop(0, NPAD // 16, zero_body, 0)

    def chunk_body(j, _):
        for i in range(CHUNK // 16):
            sidx = sall[j, pl.ds(i * 16, 16)]
            didx = dall[j, pl.ds(i * 16, 16)]
            v = plsc.load_gather(cloc, [sidx])
            plsc.addupdate_scatter(tloc, [didx], v)
        return 0

    lax.fori_loop(0, K, chunk_body, 0)
    pltpu.sync_copy(tloc, out_hbm.at[wid])


def _tc_prep_body(uy_ref, x_ref, degp_ref, a_ref, dinv_ref):
    deg = 1.0 + jnp.sum(degp_ref[...], axis=0)
    dinv = lax.rsqrt(deg)[:, None]
    latent = jnp.concatenate([uy_ref[...], x_ref[...]], axis=1)
    latent = jnp.concatenate(
        [latent, jnp.zeros((NPAD - N, D_IN), jnp.float32)], axis=0)
    a_ref[...] = latent * dinv
    dinv_ref[...] = dinv


def _tc_prep(u_Y, X, deg_parts):
    return pl.pallas_call(
        _tc_prep_body,
        out_shape=[
            jax.ShapeDtypeStruct((NPAD, D_IN), jnp.float32),
            jax.ShapeDtypeStruct((NPAD, 1), jnp.float32),
        ],
    )(u_Y, X, deg_parts)


def _tc_mlp_body(parts_ref, a_ref, dinv_ref, w1_ref, b1_ref, w2_ref, c_ref):
    i = pl.program_id(0)
    dinv = dinv_ref[...]
    z = (parts_ref[0] + parts_ref[1] + a_ref[...]) * dinv
    h = jnp.maximum(
        jnp.dot(z, w1_ref[...], preferred_element_type=jnp.float32)
        + b1_ref[...],
        0.0,
    )
    w2d = w2_ref[:, 0:1] - w2_ref[:, 1:2]
    c = jnp.dot(h, w2d, preferred_element_type=jnp.float32) * dinv
    row = i * ROWB + lax.broadcasted_iota(jnp.int32, (ROWB, 1), 0)
    c_ref[...] = jnp.where(row < N, c, 0.0)


def _tc_mlp(parts, a, dinv, w1, b1, w2):
    return pl.pallas_call(
        _tc_mlp_body,
        grid=(NBLK,),
        in_specs=[
            pl.BlockSpec((NC, ROWB, D_IN), lambda i: (0, i, 0)),
            pl.BlockSpec((ROWB, D_IN), lambda i: (i, 0)),
            pl.BlockSpec((ROWB, 1), lambda i: (i, 0)),
            pl.BlockSpec((D_IN, HID), lambda i: (0, 0)),
            pl.BlockSpec((1, HID), lambda i: (0, 0)),
            pl.BlockSpec((HID, 2), lambda i: (0, 0)),
        ],
        out_specs=pl.BlockSpec((ROWB, 1), lambda i: (i, 0)),
        out_shape=jax.ShapeDtypeStruct((NPAD, 1), jnp.float32),
    )(parts, a, dinv, w1, b1, w2)


def _tc_finish_body(tp_ref, c_ref, dinv_ref, b2_ref, out_ref):
    t = jnp.sum(tp_ref[...], axis=0)[:, None]
    delta = dinv_ref[...] * (t + c_ref[...]) + (b2_ref[0, 0] - b2_ref[0, 1])
    pos = delta >= 0.0
    ez = jnp.exp(jnp.where(pos, -delta, delta))
    p0 = jnp.where(pos, 1.0 / (1.0 + ez), ez / (1.0 + ez))
    out_ref[...] = jnp.concatenate([p0, 1.0 - p0], axis=1)


def _tc_finish(t_parts, c, dinv, b2):
    return pl.pallas_call(
        _tc_finish_body,
        grid=(NBLK,),
        in_specs=[
            pl.BlockSpec((NC * NS, ROWB), lambda i: (0, i)),
            pl.BlockSpec((ROWB, 1), lambda i: (i, 0)),
            pl.BlockSpec((ROWB, 1), lambda i: (i, 0)),
            pl.BlockSpec((1, 2), lambda i: (0, 0)),
        ],
        out_specs=pl.BlockSpec((ROWB, 2), lambda i: (i, 0)),
        out_shape=jax.ShapeDtypeStruct((NPAD, 2), jnp.float32),
    )(t_parts, c, dinv, b2)


@jax.jit
def kernel(edge_index, X, u_Y, W1, b1, W2, b2):
    src_m = edge_index[0, :EMAIN].reshape(NC, NS, KM, CHUNK)
    src_t = edge_index[0, EMAIN:].reshape(NC, NS, TAIL)
    dst_m = edge_index[1, :EMAIN].reshape(NC, NS, KM, CHUNK)
    dst_t = edge_index[1, EMAIN:].reshape(NC, NS, TAIL)

    deg_parts = _sc_degree(dst_m, dst_t)
    a, dinv = _tc_prep(u_Y, X, deg_parts)
    parts = _sc_seg128(src_m, src_t, dst_m, dst_t, a)
    c = _tc_mlp(parts, a, dinv, W1, b1.reshape(1, HID), W2)
    t_parts = _sc_segscalar(src_m, src_t, dst_m, dst_t, c.reshape(NPAD))
    out = _tc_finish(t_parts, c, dinv, b2.reshape(1, 2))
    return out[:N]


# speedup vs baseline: 1.0660x; 1.0660x over previous
import functools

import jax
import jax.numpy as jnp
from jax import lax
from jax.experimental import pallas as pl
from jax.experimental.pallas import tpu as pltpu
from jax.experimental.pallas import tpu_sc as plsc

N = 10000
E = 320000
D_IN = 128
HID = 512
NPAD = 10240
NC = 2
NS = 16
NT = NC * NS
CHUNK = 128
K = 80
KM = 78
MAIN = KM * CHUNK
TAIL = 16
EMAIN = NT * MAIN
ROWB = 512
NBLK = NPAD // ROWB
TILE_ROWS = NPAD // NS

_mesh = plsc.VectorSubcoreMesh(core_axis_name="c", subcore_axis_name="s")
_sc_params = pltpu.CompilerParams(needs_layout_passes=False)


def _fill_flat_idx(edge_hbm, row, wid, buf):
    tbase = wid * MAIN
    pltpu.sync_copy(edge_hbm.at[row, pl.ds(tbase, MAIN)],
                    buf.at[pl.ds(0, MAIN)])
    pltpu.sync_copy(edge_hbm.at[row, pl.ds(EMAIN + wid * TAIL, TAIL)],
                    buf.at[pl.ds(MAIN, TAIL)])
    lanes = lax.iota(jnp.int32, 16)
    for i in range((NPAD - N) // 16):
        buf[pl.ds(N + 16 * i, 16)] = N + 16 * i + lanes


@functools.partial(
    pl.kernel,
    out_type=jax.ShapeDtypeStruct((NT, NPAD), jnp.float32),
    mesh=_mesh,
    compiler_params=_sc_params,
    scratch_types=[
        pltpu.VMEM((NPAD,), jnp.float32),
        pltpu.VMEM((NPAD,), jnp.int32),
    ],
)
def _sc_degree(edge_hbm, out_hbm, hist, dall):
    cid = lax.axis_index("c")
    sid = lax.axis_index("s")
    wid = cid * NS + sid
    zeros16 = jnp.zeros((16,), jnp.float32)
    ones16 = jnp.ones((16,), jnp.float32)
    _fill_flat_idx(edge_hbm, 1, wid, dall)

    def zero_body(i, _):
        hist[pl.ds(i * 16, 16)] = zeros16
        return 0

    lax.fori_loop(0, NPAD // 16, zero_body, 0)

    def vec_body(i, _):
        idx = dall[pl.ds(i * 16, 16)]
        plsc.addupdate_scatter(hist, [idx], ones16)
        return 0

    lax.fori_loop(0, NPAD // 16, vec_body, 0)
    pltpu.sync_copy(hist, out_hbm.at[wid])


@functools.partial(
    pl.kernel,
    out_type=jax.ShapeDtypeStruct((NC, NPAD, D_IN), jnp.float32),
    mesh=_mesh,
    compiler_params=_sc_params,
    scratch_types=[
        pltpu.VMEM_SHARED((NPAD, D_IN), jnp.float32),
        pltpu.VMEM((CHUNK, D_IN), jnp.float32),
        pltpu.VMEM((CHUNK, D_IN), jnp.float32),
        pltpu.VMEM((4, CHUNK), jnp.int32),
        pltpu.VMEM((K, CHUNK), jnp.int32),
        pltpu.VMEM((2, CHUNK), jnp.int32),
        pltpu.SemaphoreType.DMA,
        pltpu.SemaphoreType.DMA,
        pltpu.SemaphoreType.DMA,
        pltpu.SemaphoreType.DMA,
        pltpu.SemaphoreType.DMA,
        pltpu.SemaphoreType.DMA,
        pltpu.SemaphoreType.DMA,
        pltpu.SemaphoreType.DMA,
        pltpu.SemaphoreType.DMA,
    ],
)
def _sc_seg128(edge_hbm, a_hbm, out_hbm, acc, rows0, rows1, sring, dall,
               stail, gs0, gs1, ss0, ss1, is0, is1, is2, is3, ds0):
    cid = lax.axis_index("c")
    sid = lax.axis_index("s")
    wid = cid * NS + sid
    tbase = wid * MAIN
    zeros16 = jnp.zeros((16,), jnp.float32)
    lanes = lax.iota(jnp.int32, 16)

    def dfill_start(j, _):
        pltpu.async_copy(edge_hbm.at[1, pl.ds(tbase + j * CHUNK, CHUNK)],
                         dall.at[j], ds0)
        return 0

    def dfill_wait(j, _):
        pltpu.make_async_copy(edge_hbm.at[1, pl.ds(tbase + j * CHUNK, CHUNK)],
                              dall.at[j], ds0).wait()
        return 0

    lax.fori_loop(0, KM, dfill_start, 0)
    pltpu.sync_copy(edge_hbm.at[1, pl.ds(EMAIN + wid * TAIL, TAIL)],
                    dall.at[KM, pl.ds(0, TAIL)])
    pltpu.sync_copy(edge_hbm.at[0, pl.ds(EMAIN + wid * TAIL, TAIL)],
                    stail.at[0, pl.ds(0, TAIL)])
    for i in range(7):
        dall[KM, pl.ds(TAIL + 16 * i, 16)] = N + 16 * i + lanes
        stail[0, pl.ds(TAIL + 16 * i, 16)] = N + 16 * i + lanes
    for i in range(8):
        dall[KM + 1, pl.ds(16 * i, 16)] = N + 112 + 16 * i + lanes
        stail[1, pl.ds(16 * i, 16)] = N + 112 + 16 * i + lanes
    lax.fori_loop(0, KM, dfill_wait, 0)

    def zero_rows(i, _):
        r = i // (D_IN // 16)
        col = i % (D_IN // 16)
        rows0[r, pl.ds(col * 16, 16)] = zeros16
        return 0

    lax.fori_loop(0, CHUNK * (D_IN // 16), zero_rows, 0)
    base = sid * TILE_ROWS
    for k in range(TILE_ROWS // CHUNK):
        pltpu.sync_copy(rows0, acc.at[pl.ds(base + k * CHUNK, CHUNK)])
    plsc.subcore_barrier()

    isems = [is0, is1, is2, is3]

    def idx_start(jj, slot):
        pltpu.async_copy(edge_hbm.at[0, pl.ds(tbase + jj * CHUNK, CHUNK)],
                         sring.at[slot], isems[slot])

    def idx_wait(jj, slot):
        pltpu.make_async_copy(
            edge_hbm.at[0, pl.ds(tbase + jj * CHUNK, CHUNK)],
            sring.at[slot], isems[slot]).wait()

    def gat_start(slot, rows, gsem):
        pltpu.async_copy(a_hbm.at[sring.at[slot]], rows, gsem)

    def gat_wait(slot, rows, gsem):
        pltpu.make_async_copy(a_hbm.at[sring.at[slot]], rows, gsem).wait()

    def sca_start(jj, rows, ssem):
        pltpu.async_copy(rows, acc.at[dall.at[jj]], ssem, add=True)

    def sca_wait(jj, rows, ssem):
        pltpu.make_async_copy(rows, acc.at[dall.at[jj]], ssem).wait()

    pltpu.sync_copy(edge_hbm.at[0, pl.ds(tbase, CHUNK)], sring.at[0])
    pltpu.sync_copy(edge_hbm.at[0, pl.ds(tbase + CHUNK, CHUNK)], sring.at[1])
    gat_start(0, rows0, gs0)
    gat_start(1, rows1, gs1)
    idx_start(2, 2)
    idx_start(3, 3)

    def pipe_body(it, _):
        j = 4 * it
        gat_wait(0, rows0, gs0)
        idx_start(j + 4, 0)
        sca_start(j, rows0, ss0)
        gat_wait(1, rows1, gs1)
        idx_start(j + 5, 1)
        sca_start(j + 1, rows1, ss1)
        sca_wait(j, rows0, ss0)
        idx_wait(j + 2, 2)
        gat_start(2, rows0, gs0)
        sca_wait(j + 1, rows1, ss1)
        idx_wait(j + 3, 3)
        gat_start(3, rows1, gs1)
        gat_wait(2, rows0, gs0)
        idx_start(j + 6, 2)
        sca_start(j + 2, rows0, ss0)
        gat_wait(3, rows1, gs1)
        idx_start(j + 7, 3)
        sca_start(j + 3, rows1, ss1)
        sca_wait(j + 2, rows0, ss0)
        idx_wait(j + 4, 0)
        gat_start(0, rows0, gs0)
        sca_wait(j + 3, rows1, ss1)
        idx_wait(j + 5, 1)
        gat_start(1, rows1, gs1)
        return 0

    lax.fori_loop(0, (K - 4) // 4 - 1, pipe_body, 0)

    j = K - 8
    gat_wait(0, rows0, gs0)
    idx_start(j + 4, 0)
    sca_start(j, rows0, ss0)
    gat_wait(1, rows1, gs1)
    idx_start(j + 5, 1)
    sca_start(j + 1, rows1, ss1)
    sca_wait(j, rows0, ss0)
    idx_wait(j + 2, 2)
    gat_start(2, rows0, gs0)
    sca_wait(j + 1, rows1, ss1)
    idx_wait(j + 3, 3)
    gat_start(3, rows1, gs1)
    gat_wait(2, rows0, gs0)
    sca_start(j + 2, rows0, ss0)
    gat_wait(3, rows1, gs1)
    sca_start(j + 3, rows1, ss1)
    sca_wait(j + 2, rows0, ss0)
    idx_wait(j + 4, 0)
    gat_start(0, rows0, gs0)
    sca_wait(j + 3, rows1, ss1)
    idx_wait(j + 5, 1)
    gat_start(1, rows1, gs1)

    jl = K - 4
    gat_wait(0, rows0, gs0)
    sca_start(jl, rows0, ss0)
    gat_wait(1, rows1, gs1)
    sca_start(jl + 1, rows1, ss1)
    sca_wait(jl, rows0, ss0)
    pltpu.async_copy(a_hbm.at[stail.at[0]], rows0, gs0)
    sca_wait(jl + 1, rows1, ss1)
    pltpu.async_copy(a_hbm.at[stail.at[1]], rows1, gs1)
    pltpu.make_async_copy(a_hbm.at[stail.at[0]], rows0, gs0).wait()
    sca_start(jl + 2, rows0, ss0)
    pltpu.make_async_copy(a_hbm.at[stail.at[1]], rows1, gs1).wait()
    sca_start(jl + 3, rows1, ss1)
    sca_wait(jl + 2, rows0, ss0)
    sca_wait(jl + 3, rows1, ss1)

    plsc.subcore_barrier()
    pltpu.sync_copy(acc.at[pl.ds(base, TILE_ROWS)],
                    out_hbm.at[cid, pl.ds(base, TILE_ROWS)])


@functools.partial(
    pl.kernel,
    out_type=jax.ShapeDtypeStruct((NT, NPAD), jnp.float32),
    mesh=_mesh,
    compiler_params=_sc_params,
    scratch_types=[
        pltpu.VMEM((NBLK, ROWB // CHUNK, CHUNK), jnp.float32),
        pltpu.VMEM((NPAD,), jnp.float32),
        pltpu.VMEM((NPAD,), jnp.int32),
        pltpu.VMEM((NPAD,), jnp.int32),
    ],
)
def _sc_segscalar(edge_hbm, c_hbm, out_hbm, cloc, tloc, sall, dall):
    cid = lax.axis_index("c")
    sid = lax.axis_index("s")
    wid = cid * NS + sid
    zeros16 = jnp.zeros((16,), jnp.float32)
    _fill_flat_idx(edge_hbm, 0, wid, sall)
    _fill_flat_idx(edge_hbm, 1, wid, dall)
    pltpu.sync_copy(c_hbm, cloc)

    def zero_body(i, _):
        tloc[pl.ds(i * 16, 16)] = zeros16
        return 0

    lax.fori_loop(0, NPAD // 16, zero_body, 0)

    def vec_body(i, _):
        sidx = sall[pl.ds(i * 16, 16)]
        didx = dall[pl.ds(i * 16, 16)]
        v = plsc.load_gather(
            cloc, [lax.shift_right_logical(sidx, 9),
                   lax.bitwise_and(lax.shift_right_logical(sidx, 7), 3),
                   lax.bitwise_and(sidx, 127)])
        plsc.addupdate_scatter(tloc, [didx], v)
        return 0

    lax.fori_loop(0, NPAD // 16, vec_body, 0)
    pltpu.sync_copy(tloc, out_hbm.at[wid])


def _tc_prep_body(uy_ref, x_ref, degp_ref, a_ref, dinv_ref):
    deg = 1.0 + jnp.sum(degp_ref[...], axis=0)
    dinv = lax.rsqrt(deg)[:, None]
    latent = jnp.concatenate([uy_ref[...], x_ref[...]], axis=1)
    latent = jnp.concatenate(
        [latent, jnp.zeros((NPAD - N, D_IN), jnp.float32)], axis=0)
    a_ref[...] = latent * dinv
    dinv_ref[...] = dinv


def _tc_prep(u_Y, X, deg_parts):
    return pl.pallas_call(
        _tc_prep_body,
        out_shape=[
            jax.ShapeDtypeStruct((NPAD, D_IN), jnp.float32),
            jax.ShapeDtypeStruct((NPAD, 1), jnp.float32),
        ],
    )(u_Y, X, deg_parts)


def _tc_mlp_body(parts_ref, a_ref, dinv_ref, w1_ref, b1_ref, w2_ref,
                 c_ref, c80_ref):
    i = pl.program_id(0)
    dinv = dinv_ref[...]
    z = (parts_ref[0] + parts_ref[1] + a_ref[...]) * dinv
    h = jnp.maximum(
        jnp.dot(z, w1_ref[...], preferred_element_type=jnp.float32)
        + b1_ref[...],
        0.0,
    )
    w2d = w2_ref[:, 0:1] - w2_ref[:, 1:2]
    c = jnp.dot(h, w2d, preferred_element_type=jnp.float32) * dinv
    row = i * ROWB + lax.broadcasted_iota(jnp.int32, (ROWB, 1), 0)
    c = jnp.where(row < N, c, 0.0)
    c_ref[...] = c
    c80_ref[...] = c.reshape(1, ROWB // CHUNK, CHUNK)


def _tc_mlp(parts, a, dinv, w1, b1, w2):
    return pl.pallas_call(
        _tc_mlp_body,
        grid=(NBLK,),
        in_specs=[
            pl.BlockSpec((NC, ROWB, D_IN), lambda i: (0, i, 0)),
            pl.BlockSpec((ROWB, D_IN), lambda i: (i, 0)),
            pl.BlockSpec((ROWB, 1), lambda i: (i, 0)),
            pl.BlockSpec((D_IN, HID), lambda i: (0, 0)),
            pl.BlockSpec((1, HID), lambda i: (0, 0)),
            pl.BlockSpec((HID, 2), lambda i: (0, 0)),
        ],
        out_specs=[
            pl.BlockSpec((ROWB, 1), lambda i: (i, 0)),
            pl.BlockSpec((1, ROWB // CHUNK, CHUNK), lambda i: (i, 0, 0)),
        ],
        out_shape=[
            jax.ShapeDtypeStruct((NPAD, 1), jnp.float32),
            jax.ShapeDtypeStruct((NBLK, ROWB // CHUNK, CHUNK), jnp.float32),
        ],
    )(parts, a, dinv, w1, b1, w2)


def _tc_finish_body(tp_ref, c_ref, dinv_ref, b2_ref, out_ref):
    t = jnp.sum(tp_ref[...], axis=0)[:, None]
    delta = dinv_ref[...] * (t + c_ref[...]) + (b2_ref[0, 0] - b2_ref[0, 1])
    pos = delta >= 0.0
    ez = jnp.exp(jnp.where(pos, -delta, delta))
    p0 = jnp.where(pos, 1.0 / (1.0 + ez), ez / (1.0 + ez))
    out_ref[...] = jnp.concatenate([p0, 1.0 - p0], axis=1)


def _tc_finish(t_parts, c, dinv, b2):
    return pl.pallas_call(
        _tc_finish_body,
        grid=(NBLK,),
        in_specs=[
            pl.BlockSpec((NT, ROWB), lambda i: (0, i)),
            pl.BlockSpec((ROWB, 1), lambda i: (i, 0)),
            pl.BlockSpec((ROWB, 1), lambda i: (i, 0)),
            pl.BlockSpec((1, 2), lambda i: (0, 0)),
        ],
        out_specs=pl.BlockSpec((ROWB, 2), lambda i: (i, 0)),
        out_shape=jax.ShapeDtypeStruct((NPAD, 2), jnp.float32),
    )(t_parts, c, dinv, b2)


@jax.jit
def kernel(edge_index, X, u_Y, W1, b1, W2, b2):
    deg_parts = _sc_degree(edge_index)
    a, dinv = _tc_prep(u_Y, X, deg_parts)
    parts = _sc_seg128(edge_index, a)
    c, c80 = _tc_mlp(parts, a, dinv, W1, b1.reshape(1, HID), W2)
    t_parts = _sc_segscalar(edge_index, c80)
    out = _tc_finish(t_parts, c, dinv, b2.reshape(1, 2))
    return out[:N]


# speedup vs baseline: 1.0807x; 1.0137x over previous
import functools

import jax
import jax.numpy as jnp
from jax import lax
from jax.experimental import pallas as pl
from jax.experimental.pallas import tpu as pltpu
from jax.experimental.pallas import tpu_sc as plsc

N = 10000
E = 320000
D_IN = 128
HID = 512
NPAD = 10240
NC = 2
NS = 16
NT = NC * NS
CHUNK = 128
K = 80
KM = 78
MAIN = KM * CHUNK
TAIL = 16
EMAIN = NT * MAIN
ROWB = 512
NBLK = NPAD // ROWB
TILE_ROWS = NPAD // NS

_mesh = plsc.VectorSubcoreMesh(core_axis_name="c", subcore_axis_name="s")
_sc_params = pltpu.CompilerParams(needs_layout_passes=False)


def _fill_flat_idx(edge_hbm, row, wid, buf):
    tbase = wid * MAIN
    pltpu.sync_copy(edge_hbm.at[row, pl.ds(tbase, MAIN)],
                    buf.at[pl.ds(0, MAIN)])
    pltpu.sync_copy(edge_hbm.at[row, pl.ds(EMAIN + wid * TAIL, TAIL)],
                    buf.at[pl.ds(MAIN, TAIL)])
    lanes = lax.iota(jnp.int32, 16)
    for i in range((NPAD - N) // 16):
        buf[pl.ds(N + 16 * i, 16)] = N + 16 * i + lanes


@functools.partial(
    pl.kernel,
    out_type=jax.ShapeDtypeStruct((NT, NPAD), jnp.float32),
    mesh=_mesh,
    compiler_params=_sc_params,
    scratch_types=[
        pltpu.VMEM((NPAD,), jnp.float32),
        pltpu.VMEM((NPAD,), jnp.int32),
    ],
)
def _sc_degree(edge_hbm, out_hbm, hist, dall):
    cid = lax.axis_index("c")
    sid = lax.axis_index("s")
    wid = cid * NS + sid
    zeros16 = jnp.zeros((16,), jnp.float32)
    ones16 = jnp.ones((16,), jnp.float32)
    _fill_flat_idx(edge_hbm, 1, wid, dall)

    def zero_body(i, _):
        hist[pl.ds(i * 16, 16)] = zeros16
        return 0

    lax.fori_loop(0, NPAD // 16, zero_body, 0)

    def vec_body(i, _):
        idx = dall[pl.ds(i * 16, 16)]
        plsc.addupdate_scatter(hist, [idx], ones16)
        return 0

    lax.fori_loop(0, NPAD // 16, vec_body, 0)
    pltpu.sync_copy(hist, out_hbm.at[wid])


@functools.partial(
    pl.kernel,
    out_type=jax.ShapeDtypeStruct((NC, NPAD, D_IN), jnp.float32),
    mesh=_mesh,
    compiler_params=_sc_params,
    scratch_types=[
        pltpu.VMEM_SHARED((NPAD, D_IN), jnp.float32),
        pltpu.VMEM((CHUNK, D_IN), jnp.float32),
        pltpu.VMEM((CHUNK, D_IN), jnp.float32),
        pltpu.VMEM((4, CHUNK), jnp.int32),
        pltpu.VMEM((K, CHUNK), jnp.int32),
        pltpu.VMEM((2, CHUNK), jnp.int32),
        pltpu.SemaphoreType.DMA,
        pltpu.SemaphoreType.DMA,
        pltpu.SemaphoreType.DMA,
        pltpu.SemaphoreType.DMA,
        pltpu.SemaphoreType.DMA,
        pltpu.SemaphoreType.DMA,
        pltpu.SemaphoreType.DMA,
        pltpu.SemaphoreType.DMA,
        pltpu.SemaphoreType.DMA,
    ],
)
def _sc_seg128(edge_hbm, a_hbm, out_hbm, acc, rows0, rows1, sring, dall,
               stail, gs0, gs1, ss0, ss1, is0, is1, is2, is3, ds0):
    cid = lax.axis_index("c")
    sid = lax.axis_index("s")
    wid = cid * NS + sid
    tbase = wid * MAIN
    zeros16 = jnp.zeros((16,), jnp.float32)
    lanes = lax.iota(jnp.int32, 16)

    def dfill_start(j, _):
        pltpu.async_copy(edge_hbm.at[1, pl.ds(tbase + j * CHUNK, CHUNK)],
                         dall.at[j], ds0)
        return 0

    def dfill_wait(j, _):
        pltpu.make_async_copy(edge_hbm.at[1, pl.ds(tbase + j * CHUNK, CHUNK)],
                              dall.at[j], ds0).wait()
        return 0

    lax.fori_loop(0, KM, dfill_start, 0)
    pltpu.sync_copy(edge_hbm.at[1, pl.ds(EMAIN + wid * TAIL, TAIL)],
                    dall.at[KM, pl.ds(0, TAIL)])
    pltpu.sync_copy(edge_hbm.at[0, pl.ds(EMAIN + wid * TAIL, TAIL)],
                    stail.at[0, pl.ds(0, TAIL)])
    for i in range(7):
        dall[KM, pl.ds(TAIL + 16 * i, 16)] = N + 16 * i + lanes
        stail[0, pl.ds(TAIL + 16 * i, 16)] = N + 16 * i + lanes
    for i in range(8):
        dall[KM + 1, pl.ds(16 * i, 16)] = N + 112 + 16 * i + lanes
        stail[1, pl.ds(16 * i, 16)] = N + 112 + 16 * i + lanes
    lax.fori_loop(0, KM, dfill_wait, 0)

    def zero_rows(i, _):
        r = i // (D_IN // 16)
        col = i % (D_IN // 16)
        rows0[r, pl.ds(col * 16, 16)] = zeros16
        return 0

    lax.fori_loop(0, CHUNK * (D_IN // 16), zero_rows, 0)
    base = sid * TILE_ROWS
    for k in range(TILE_ROWS // CHUNK):
        pltpu.sync_copy(rows0, acc.at[pl.ds(base + k * CHUNK, CHUNK)])
    plsc.subcore_barrier()

    isems = [is0, is1, is2, is3]

    def idx_start(jj, slot):
        pltpu.async_copy(edge_hbm.at[0, pl.ds(tbase + jj * CHUNK, CHUNK)],
                         sring.at[slot], isems[slot])

    def idx_wait(jj, slot):
        pltpu.make_async_copy(
            edge_hbm.at[0, pl.ds(tbase + jj * CHUNK, CHUNK)],
            sring.at[slot], isems[slot]).wait()

    def gat_start(slot, rows, gsem):
        pltpu.async_copy(a_hbm.at[sring.at[slot]], rows, gsem)

    def gat_wait(slot, rows, gsem):
        pltpu.make_async_copy(a_hbm.at[sring.at[slot]], rows, gsem).wait()

    def sca_start(jj, rows, ssem):
        pltpu.async_copy(rows, acc.at[dall.at[jj]], ssem, add=True)

    def sca_wait(jj, rows, ssem):
        pltpu.make_async_copy(rows, acc.at[dall.at[jj]], ssem).wait()

    pltpu.sync_copy(edge_hbm.at[0, pl.ds(tbase, CHUNK)], sring.at[0])
    pltpu.sync_copy(edge_hbm.at[0, pl.ds(tbase + CHUNK, CHUNK)], sring.at[1])
    gat_start(0, rows0, gs0)
    gat_start(1, rows1, gs1)
    idx_start(2, 2)
    idx_start(3, 3)

    def pipe_body(it, _):
        j = 4 * it
        gat_wait(0, rows0, gs0)
        idx_start(j + 4, 0)
        sca_start(j, rows0, ss0)
        gat_wait(1, rows1, gs1)
        idx_start(j + 5, 1)
        sca_start(j + 1, rows1, ss1)
        sca_wait(j, rows0, ss0)
        idx_wait(j + 2, 2)
        gat_start(2, rows0, gs0)
        sca_wait(j + 1, rows1, ss1)
        idx_wait(j + 3, 3)
        gat_start(3, rows1, gs1)
        gat_wait(2, rows0, gs0)
        idx_start(j + 6, 2)
        sca_start(j + 2, rows0, ss0)
        gat_wait(3, rows1, gs1)
        idx_start(j + 7, 3)
        sca_start(j + 3, rows1, ss1)
        sca_wait(j + 2, rows0, ss0)
        idx_wait(j + 4, 0)
        gat_start(0, rows0, gs0)
        sca_wait(j + 3, rows1, ss1)
        idx_wait(j + 5, 1)
        gat_start(1, rows1, gs1)
        return 0

    lax.fori_loop(0, (K - 4) // 4 - 1, pipe_body, 0)

    j = K - 8
    gat_wait(0, rows0, gs0)
    idx_start(j + 4, 0)
    sca_start(j, rows0, ss0)
    gat_wait(1, rows1, gs1)
    idx_start(j + 5, 1)
    sca_start(j + 1, rows1, ss1)
    sca_wait(j, rows0, ss0)
    idx_wait(j + 2, 2)
    gat_start(2, rows0, gs0)
    sca_wait(j + 1, rows1, ss1)
    idx_wait(j + 3, 3)
    gat_start(3, rows1, gs1)
    gat_wait(2, rows0, gs0)
    sca_start(j + 2, rows0, ss0)
    gat_wait(3, rows1, gs1)
    sca_start(j + 3, rows1, ss1)
    sca_wait(j + 2, rows0, ss0)
    idx_wait(j + 4, 0)
    gat_start(0, rows0, gs0)
    sca_wait(j + 3, rows1, ss1)
    idx_wait(j + 5, 1)
    gat_start(1, rows1, gs1)

    jl = K - 4
    gat_wait(0, rows0, gs0)
    sca_start(jl, rows0, ss0)
    gat_wait(1, rows1, gs1)
    sca_start(jl + 1, rows1, ss1)
    sca_wait(jl, rows0, ss0)
    pltpu.async_copy(a_hbm.at[stail.at[0]], rows0, gs0)
    sca_wait(jl + 1, rows1, ss1)
    pltpu.async_copy(a_hbm.at[stail.at[1]], rows1, gs1)
    pltpu.make_async_copy(a_hbm.at[stail.at[0]], rows0, gs0).wait()
    sca_start(jl + 2, rows0, ss0)
    pltpu.make_async_copy(a_hbm.at[stail.at[1]], rows1, gs1).wait()
    sca_start(jl + 3, rows1, ss1)
    sca_wait(jl + 2, rows0, ss0)
    sca_wait(jl + 3, rows1, ss1)

    plsc.subcore_barrier()
    pltpu.sync_copy(acc.at[pl.ds(base, TILE_ROWS)],
                    out_hbm.at[cid, pl.ds(base, TILE_ROWS)])


@functools.partial(
    pl.kernel,
    out_type=jax.ShapeDtypeStruct((NT, NPAD), jnp.float32),
    mesh=_mesh,
    compiler_params=_sc_params,
    scratch_types=[
        pltpu.VMEM((NBLK, ROWB // CHUNK, CHUNK), jnp.float32),
        pltpu.VMEM((NPAD,), jnp.float32),
        pltpu.VMEM((NPAD,), jnp.int32),
        pltpu.VMEM((NPAD,), jnp.int32),
    ],
)
def _sc_segscalar(edge_hbm, c_hbm, out_hbm, cloc, tloc, sall, dall):
    cid = lax.axis_index("c")
    sid = lax.axis_index("s")
    wid = cid * NS + sid
    zeros16 = jnp.zeros((16,), jnp.float32)
    _fill_flat_idx(edge_hbm, 0, wid, sall)
    _fill_flat_idx(edge_hbm, 1, wid, dall)
    pltpu.sync_copy(c_hbm, cloc)

    def zero_body(i, _):
        tloc[pl.ds(i * 16, 16)] = zeros16
        return 0

    lax.fori_loop(0, NPAD // 16, zero_body, 0)

    def vec_body(i, _):
        sidx = sall[pl.ds(i * 16, 16)]
        didx = dall[pl.ds(i * 16, 16)]
        v = plsc.load_gather(
            cloc, [lax.shift_right_logical(sidx, 9),
                   lax.bitwise_and(lax.shift_right_logical(sidx, 7), 3),
                   lax.bitwise_and(sidx, 127)])
        plsc.addupdate_scatter(tloc, [didx], v)
        return 0

    lax.fori_loop(0, NPAD // 16, vec_body, 0)
    pltpu.sync_copy(tloc, out_hbm.at[wid])


def _tc_prep_body(uy_ref, x_ref, degp_ref, a_ref, dinv_ref):
    deg = 1.0 + jnp.sum(degp_ref[...], axis=0)
    dinv = lax.rsqrt(deg)[:, None]
    latent = jnp.concatenate([uy_ref[...], x_ref[...]], axis=1)
    latent = jnp.concatenate(
        [latent, jnp.zeros((NPAD - N, D_IN), jnp.float32)], axis=0)
    a_ref[...] = latent * dinv
    dinv_ref[...] = dinv


def _tc_prep(u_Y, X, deg_parts):
    return pl.pallas_call(
        _tc_prep_body,
        out_shape=[
            jax.ShapeDtypeStruct((NPAD, D_IN), jnp.float32),
            jax.ShapeDtypeStruct((NPAD, 1), jnp.float32),
        ],
    )(u_Y, X, deg_parts)


def _tc_mlp_body(parts_ref, a_ref, dinv_ref, w1_ref, b1_ref, w2_ref,
                 c80_ref):
    i = pl.program_id(0)
    dinv = dinv_ref[...]
    z = (parts_ref[0] + parts_ref[1] + a_ref[...]) * dinv
    h = jnp.maximum(
        jnp.dot(z, w1_ref[...], preferred_element_type=jnp.float32)
        + b1_ref[...],
        0.0,
    )
    w2d = w2_ref[:, 0:1] - w2_ref[:, 1:2]
    c = jnp.dot(h, w2d, preferred_element_type=jnp.float32) * dinv
    row = i * ROWB + lax.broadcasted_iota(jnp.int32, (ROWB, 1), 0)
    c = jnp.where(row < N, c, 0.0)
    c80_ref[...] = c.reshape(1, ROWB // CHUNK, CHUNK)


def _tc_mlp(parts, a, dinv, w1, b1, w2):
    return pl.pallas_call(
        _tc_mlp_body,
        grid=(NBLK,),
        in_specs=[
            pl.BlockSpec((NC, ROWB, D_IN), lambda i: (0, i, 0)),
            pl.BlockSpec((ROWB, D_IN), lambda i: (i, 0)),
            pl.BlockSpec((ROWB, 1), lambda i: (i, 0)),
            pl.BlockSpec((D_IN, HID), lambda i: (0, 0)),
            pl.BlockSpec((1, HID), lambda i: (0, 0)),
            pl.BlockSpec((HID, 2), lambda i: (0, 0)),
        ],
        out_specs=pl.BlockSpec((1, ROWB // CHUNK, CHUNK),
                               lambda i: (i, 0, 0)),
        out_shape=jax.ShapeDtypeStruct((NBLK, ROWB // CHUNK, CHUNK),
                                       jnp.float32),
    )(parts, a, dinv, w1, b1, w2)


def _tc_finish_body(tp_ref, c80_ref, dinv_ref, b2_ref, out_ref):
    t = jnp.sum(tp_ref[...], axis=0).reshape(ROWB // CHUNK, CHUNK)
    dinv = dinv_ref[...].reshape(ROWB // CHUNK, CHUNK)
    delta = dinv * (t + c80_ref[0]) + (b2_ref[0, 0] - b2_ref[0, 1])
    pos = delta >= 0.0
    ez = jnp.exp(jnp.where(pos, -delta, delta))
    p0 = jnp.where(pos, 1.0 / (1.0 + ez), ez / (1.0 + ez))
    out_ref[...] = jnp.concatenate(
        [p0.reshape(ROWB, 1), (1.0 - p0).reshape(ROWB, 1)], axis=1)


def _tc_finish(t_parts, c80, dinv, b2):
    return pl.pallas_call(
        _tc_finish_body,
        grid=(NBLK,),
        in_specs=[
            pl.BlockSpec((NT, ROWB), lambda i: (0, i)),
            pl.BlockSpec((1, ROWB // CHUNK, CHUNK), lambda i: (i, 0, 0)),
            pl.BlockSpec((ROWB, 1), lambda i: (i, 0)),
            pl.BlockSpec((1, 2), lambda i: (0, 0)),
        ],
        out_specs=pl.BlockSpec((ROWB, 2), lambda i: (i, 0)),
        out_shape=jax.ShapeDtypeStruct((NPAD, 2), jnp.float32),
    )(t_parts, c80, dinv, b2)


@jax.jit
def kernel(edge_index, X, u_Y, W1, b1, W2, b2):
    deg_parts = _sc_degree(edge_index)
    a, dinv = _tc_prep(u_Y, X, deg_parts)
    parts = _sc_seg128(edge_index, a)
    c80 = _tc_mlp(parts, a, dinv, W1, b1.reshape(1, HID), W2)
    t_parts = _sc_segscalar(edge_index, c80)
    out = _tc_finish(t_parts, c80, dinv, b2.reshape(1, 2))
    return out[:N]


# speedup vs baseline: 1.1158x; 1.0325x over previous
import functools

import jax
import jax.numpy as jnp
from jax import lax
from jax.experimental import pallas as pl
from jax.experimental.pallas import tpu as pltpu
from jax.experimental.pallas import tpu_sc as plsc

N = 10000
E = 320000
D_IN = 128
HID = 512
NPAD = 10240
NC = 2
NS = 16
NT = NC * NS
CHUNK = 128
K = 80
KM = 78
MAIN = KM * CHUNK
TAIL = 16
EMAIN = NT * MAIN
ROWB = 512
NBLK = NPAD // ROWB
TILE_ROWS = NPAD // NS

_mesh = plsc.VectorSubcoreMesh(core_axis_name="c", subcore_axis_name="s")
_sc_params = pltpu.CompilerParams(needs_layout_passes=False)


def _fill_flat_idx(edge_hbm, row, wid, buf):
    tbase = wid * MAIN
    pltpu.sync_copy(edge_hbm.at[row, pl.ds(tbase, MAIN)],
                    buf.at[pl.ds(0, MAIN)])
    pltpu.sync_copy(edge_hbm.at[row, pl.ds(EMAIN + wid * TAIL, TAIL)],
                    buf.at[pl.ds(MAIN, TAIL)])
    lanes = lax.iota(jnp.int32, 16)
    for i in range((NPAD - N) // 16):
        buf[pl.ds(N + 16 * i, 16)] = N + 16 * i + lanes


@functools.partial(
    pl.kernel,
    out_type=jax.ShapeDtypeStruct((NT, NPAD), jnp.float32),
    mesh=_mesh,
    compiler_params=_sc_params,
    scratch_types=[
        pltpu.VMEM((NPAD,), jnp.float32),
        pltpu.VMEM((NPAD,), jnp.int32),
    ],
)
def _sc_degree(edge_hbm, out_hbm, hist, dall):
    cid = lax.axis_index("c")
    sid = lax.axis_index("s")
    wid = cid * NS + sid
    zeros16 = jnp.zeros((16,), jnp.float32)
    ones16 = jnp.ones((16,), jnp.float32)
    _fill_flat_idx(edge_hbm, 1, wid, dall)

    def zero_body(i, _):
        for u in range(4):
            hist[pl.ds(i * 64 + u * 16, 16)] = zeros16
        return 0

    lax.fori_loop(0, NPAD // 64, zero_body, 0)

    def vec_body(i, _):
        for u in range(4):
            idx = dall[pl.ds(i * 64 + u * 16, 16)]
            plsc.addupdate_scatter(hist, [idx], ones16)
        return 0

    lax.fori_loop(0, NPAD // 64, vec_body, 0)
    pltpu.sync_copy(hist, out_hbm.at[wid])


@functools.partial(
    pl.kernel,
    out_type=jax.ShapeDtypeStruct((NC, NPAD, D_IN), jnp.float32),
    mesh=_mesh,
    compiler_params=_sc_params,
    scratch_types=[
        pltpu.VMEM_SHARED((NPAD, D_IN), jnp.float32),
        pltpu.VMEM((CHUNK, D_IN), jnp.float32),
        pltpu.VMEM((CHUNK, D_IN), jnp.float32),
        pltpu.VMEM((4, CHUNK), jnp.int32),
        pltpu.VMEM((K, CHUNK), jnp.int32),
        pltpu.VMEM((2, CHUNK), jnp.int32),
        pltpu.SemaphoreType.DMA,
        pltpu.SemaphoreType.DMA,
        pltpu.SemaphoreType.DMA,
        pltpu.SemaphoreType.DMA,
        pltpu.SemaphoreType.DMA,
        pltpu.SemaphoreType.DMA,
        pltpu.SemaphoreType.DMA,
        pltpu.SemaphoreType.DMA,
        pltpu.SemaphoreType.DMA,
    ],
)
def _sc_seg128(edge_hbm, a_hbm, out_hbm, acc, rows0, rows1, sring, dall,
               stail, gs0, gs1, ss0, ss1, is0, is1, is2, is3, ds0):
    cid = lax.axis_index("c")
    sid = lax.axis_index("s")
    wid = cid * NS + sid
    tbase = wid * MAIN
    zeros16 = jnp.zeros((16,), jnp.float32)
    lanes = lax.iota(jnp.int32, 16)

    def dfill_start(j, _):
        pltpu.async_copy(edge_hbm.at[1, pl.ds(tbase + j * CHUNK, CHUNK)],
                         dall.at[j], ds0)
        return 0

    def dfill_wait(j, _):
        pltpu.make_async_copy(edge_hbm.at[1, pl.ds(tbase + j * CHUNK, CHUNK)],
                              dall.at[j], ds0).wait()
        return 0

    lax.fori_loop(0, KM, dfill_start, 0)
    pltpu.sync_copy(edge_hbm.at[1, pl.ds(EMAIN + wid * TAIL, TAIL)],
                    dall.at[KM, pl.ds(0, TAIL)])
    pltpu.sync_copy(edge_hbm.at[0, pl.ds(EMAIN + wid * TAIL, TAIL)],
                    stail.at[0, pl.ds(0, TAIL)])
    for i in range(7):
        dall[KM, pl.ds(TAIL + 16 * i, 16)] = N + 16 * i + lanes
        stail[0, pl.ds(TAIL + 16 * i, 16)] = N + 16 * i + lanes
    for i in range(8):
        dall[KM + 1, pl.ds(16 * i, 16)] = N + 112 + 16 * i + lanes
        stail[1, pl.ds(16 * i, 16)] = N + 112 + 16 * i + lanes
    lax.fori_loop(0, KM, dfill_wait, 0)

    def zero_rows(r, _):
        for col in range(D_IN // 16):
            rows0[r, pl.ds(col * 16, 16)] = zeros16
        return 0

    lax.fori_loop(0, CHUNK, zero_rows, 0)
    base = sid * TILE_ROWS
    for k in range(TILE_ROWS // CHUNK):
        pltpu.sync_copy(rows0, acc.at[pl.ds(base + k * CHUNK, CHUNK)])
    plsc.subcore_barrier()

    isems = [is0, is1, is2, is3]

    def idx_start(jj, slot):
        pltpu.async_copy(edge_hbm.at[0, pl.ds(tbase + jj * CHUNK, CHUNK)],
                         sring.at[slot], isems[slot])

    def idx_wait(jj, slot):
        pltpu.make_async_copy(
            edge_hbm.at[0, pl.ds(tbase + jj * CHUNK, CHUNK)],
            sring.at[slot], isems[slot]).wait()

    def gat_start(slot, rows, gsem):
        pltpu.async_copy(a_hbm.at[sring.at[slot]], rows, gsem)

    def gat_wait(slot, rows, gsem):
        pltpu.make_async_copy(a_hbm.at[sring.at[slot]], rows, gsem).wait()

    def sca_start(jj, rows, ssem):
        pltpu.async_copy(rows, acc.at[dall.at[jj]], ssem, add=True)

    def sca_wait(jj, rows, ssem):
        pltpu.make_async_copy(rows, acc.at[dall.at[jj]], ssem).wait()

    pltpu.sync_copy(edge_hbm.at[0, pl.ds(tbase, CHUNK)], sring.at[0])
    pltpu.sync_copy(edge_hbm.at[0, pl.ds(tbase + CHUNK, CHUNK)], sring.at[1])
    gat_start(0, rows0, gs0)
    gat_start(1, rows1, gs1)
    idx_start(2, 2)
    idx_start(3, 3)

    def pipe_body(it, _):
        j = 4 * it
        gat_wait(0, rows0, gs0)
        idx_start(j + 4, 0)
        sca_start(j, rows0, ss0)
        gat_wait(1, rows1, gs1)
        idx_start(j + 5, 1)
        sca_start(j + 1, rows1, ss1)
        sca_wait(j, rows0, ss0)
        idx_wait(j + 2, 2)
        gat_start(2, rows0, gs0)
        sca_wait(j + 1, rows1, ss1)
        idx_wait(j + 3, 3)
        gat_start(3, rows1, gs1)
        gat_wait(2, rows0, gs0)
        idx_start(j + 6, 2)
        sca_start(j + 2, rows0, ss0)
        gat_wait(3, rows1, gs1)
        idx_start(j + 7, 3)
        sca_start(j + 3, rows1, ss1)
        sca_wait(j + 2, rows0, ss0)
        idx_wait(j + 4, 0)
        gat_start(0, rows0, gs0)
        sca_wait(j + 3, rows1, ss1)
        idx_wait(j + 5, 1)
        gat_start(1, rows1, gs1)
        return 0

    lax.fori_loop(0, (K - 4) // 4 - 1, pipe_body, 0)

    j = K - 8
    gat_wait(0, rows0, gs0)
    idx_start(j + 4, 0)
    sca_start(j, rows0, ss0)
    gat_wait(1, rows1, gs1)
    idx_start(j + 5, 1)
    sca_start(j + 1, rows1, ss1)
    sca_wait(j, rows0, ss0)
    idx_wait(j + 2, 2)
    gat_start(2, rows0, gs0)
    sca_wait(j + 1, rows1, ss1)
    idx_wait(j + 3, 3)
    gat_start(3, rows1, gs1)
    gat_wait(2, rows0, gs0)
    sca_start(j + 2, rows0, ss0)
    gat_wait(3, rows1, gs1)
    sca_start(j + 3, rows1, ss1)
    sca_wait(j + 2, rows0, ss0)
    idx_wait(j + 4, 0)
    gat_start(0, rows0, gs0)
    sca_wait(j + 3, rows1, ss1)
    idx_wait(j + 5, 1)
    gat_start(1, rows1, gs1)

    jl = K - 4
    gat_wait(0, rows0, gs0)
    sca_start(jl, rows0, ss0)
    gat_wait(1, rows1, gs1)
    sca_start(jl + 1, rows1, ss1)
    sca_wait(jl, rows0, ss0)
    pltpu.async_copy(a_hbm.at[stail.at[0]], rows0, gs0)
    sca_wait(jl + 1, rows1, ss1)
    pltpu.async_copy(a_hbm.at[stail.at[1]], rows1, gs1)
    pltpu.make_async_copy(a_hbm.at[stail.at[0]], rows0, gs0).wait()
    sca_start(jl + 2, rows0, ss0)
    pltpu.make_async_copy(a_hbm.at[stail.at[1]], rows1, gs1).wait()
    sca_start(jl + 3, rows1, ss1)
    sca_wait(jl + 2, rows0, ss0)
    sca_wait(jl + 3, rows1, ss1)

    plsc.subcore_barrier()
    pltpu.sync_copy(acc.at[pl.ds(base, TILE_ROWS)],
                    out_hbm.at[cid, pl.ds(base, TILE_ROWS)])


@functools.partial(
    pl.kernel,
    out_type=jax.ShapeDtypeStruct((NT, NPAD), jnp.float32),
    mesh=_mesh,
    compiler_params=_sc_params,
    scratch_types=[
        pltpu.VMEM((NBLK, ROWB // CHUNK, CHUNK), jnp.float32),
        pltpu.VMEM((NPAD,), jnp.float32),
        pltpu.VMEM((NPAD,), jnp.int32),
        pltpu.VMEM((NPAD,), jnp.int32),
    ],
)
def _sc_segscalar(edge_hbm, c_hbm, out_hbm, cloc, tloc, sall, dall):
    cid = lax.axis_index("c")
    sid = lax.axis_index("s")
    wid = cid * NS + sid
    zeros16 = jnp.zeros((16,), jnp.float32)
    _fill_flat_idx(edge_hbm, 0, wid, sall)
    _fill_flat_idx(edge_hbm, 1, wid, dall)
    pltpu.sync_copy(c_hbm, cloc)

    def zero_body(i, _):
        for u in range(4):
            tloc[pl.ds(i * 64 + u * 16, 16)] = zeros16
        return 0

    lax.fori_loop(0, NPAD // 64, zero_body, 0)

    def vec_body(i, _):
        for u in range(4):
            sidx = sall[pl.ds(i * 64 + u * 16, 16)]
            didx = dall[pl.ds(i * 64 + u * 16, 16)]
            v = plsc.load_gather(
                cloc, [lax.shift_right_logical(sidx, 9),
                       lax.bitwise_and(lax.shift_right_logical(sidx, 7), 3),
                       lax.bitwise_and(sidx, 127)])
            plsc.addupdate_scatter(tloc, [didx], v)
        return 0

    lax.fori_loop(0, NPAD // 64, vec_body, 0)
    pltpu.sync_copy(tloc, out_hbm.at[wid])


def _tc_prep_body(uy_ref, x_ref, degp_ref, a_ref, dinv_ref):
    deg = 1.0 + jnp.sum(degp_ref[...], axis=0)
    dinv = lax.rsqrt(deg)[:, None]
    latent = jnp.concatenate([uy_ref[...], x_ref[...]], axis=1)
    latent = jnp.concatenate(
        [latent, jnp.zeros((NPAD - N, D_IN), jnp.float32)], axis=0)
    a_ref[...] = latent * dinv
    dinv_ref[...] = dinv


def _tc_prep(u_Y, X, deg_parts):
    return pl.pallas_call(
        _tc_prep_body,
        out_shape=[
            jax.ShapeDtypeStruct((NPAD, D_IN), jnp.float32),
            jax.ShapeDtypeStruct((NPAD, 1), jnp.float32),
        ],
    )(u_Y, X, deg_parts)


def _tc_mlp_body(parts_ref, a_ref, dinv_ref, w1_ref, b1_ref, w2_ref,
                 c80_ref):
    i = pl.program_id(0)
    dinv = dinv_ref[...]
    z = (parts_ref[0] + parts_ref[1] + a_ref[...]) * dinv
    h = jnp.maximum(
        jnp.dot(z, w1_ref[...], preferred_element_type=jnp.float32)
        + b1_ref[...],
        0.0,
    )
    w2d = w2_ref[:, 0:1] - w2_ref[:, 1:2]
    c = jnp.dot(h, w2d, preferred_element_type=jnp.float32) * dinv
    row = i * ROWB + lax.broadcasted_iota(jnp.int32, (ROWB, 1), 0)
    c = jnp.where(row < N, c, 0.0)
    c80_ref[...] = c.reshape(1, ROWB // CHUNK, CHUNK)


def _tc_mlp(parts, a, dinv, w1, b1, w2):
    return pl.pallas_call(
        _tc_mlp_body,
        grid=(NBLK,),
        in_specs=[
            pl.BlockSpec((NC, ROWB, D_IN), lambda i: (0, i, 0)),
            pl.BlockSpec((ROWB, D_IN), lambda i: (i, 0)),
            pl.BlockSpec((ROWB, 1), lambda i: (i, 0)),
            pl.BlockSpec((D_IN, HID), lambda i: (0, 0)),
            pl.BlockSpec((1, HID), lambda i: (0, 0)),
            pl.BlockSpec((HID, 2), lambda i: (0, 0)),
        ],
        out_specs=pl.BlockSpec((1, ROWB // CHUNK, CHUNK),
                               lambda i: (i, 0, 0)),
        out_shape=jax.ShapeDtypeStruct((NBLK, ROWB // CHUNK, CHUNK),
                                       jnp.float32),
    )(parts, a, dinv, w1, b1, w2)


def _tc_finish_body(tp_ref, c80_ref, dinv_ref, b2_ref, out_ref):
    t = jnp.sum(tp_ref[...], axis=0).reshape(ROWB // CHUNK, CHUNK)
    dinv = dinv_ref[...].reshape(ROWB // CHUNK, CHUNK)
    delta = dinv * (t + c80_ref[0]) + (b2_ref[0, 0] - b2_ref[0, 1])
    pos = delta >= 0.0
    ez = jnp.exp(jnp.where(pos, -delta, delta))
    p0 = jnp.where(pos, 1.0 / (1.0 + ez), ez / (1.0 + ez))
    out_ref[...] = jnp.concatenate(
        [p0.reshape(ROWB, 1), (1.0 - p0).reshape(ROWB, 1)], axis=1)


def _tc_finish(t_parts, c80, dinv, b2):
    return pl.pallas_call(
        _tc_finish_body,
        grid=(NBLK,),
        in_specs=[
            pl.BlockSpec((NT, ROWB), lambda i: (0, i)),
            pl.BlockSpec((1, ROWB // CHUNK, CHUNK), lambda i: (i, 0, 0)),
            pl.BlockSpec((ROWB, 1), lambda i: (i, 0)),
            pl.BlockSpec((1, 2), lambda i: (0, 0)),
        ],
        out_specs=pl.BlockSpec((ROWB, 2), lambda i: (i, 0)),
        out_shape=jax.ShapeDtypeStruct((NPAD, 2), jnp.float32),
    )(t_parts, c80, dinv, b2)


@jax.jit
def kernel(edge_index, X, u_Y, W1, b1, W2, b2):
    deg_parts = _sc_degree(edge_index)
    a, dinv = _tc_prep(u_Y, X, deg_parts)
    parts = _sc_seg128(edge_index, a)
    c80 = _tc_mlp(parts, a, dinv, W1, b1.reshape(1, HID), W2)
    t_parts = _sc_segscalar(edge_index, c80)
    out = _tc_finish(t_parts, c80, dinv, b2.reshape(1, 2))
    return out[:N]


# speedup vs baseline: 1.1605x; 1.0401x over previous
import functools

import jax
import jax.numpy as jnp
from jax import lax
from jax.experimental import pallas as pl
from jax.experimental.pallas import tpu as pltpu
from jax.experimental.pallas import tpu_sc as plsc

N = 10000
E = 320000
D_IN = 128
HID = 512
NPAD = 10240
NC = 2
NS = 16
NT = NC * NS
CHUNK = 128
K = 80
KM = 78
MAIN = KM * CHUNK
TAIL = 16
EMAIN = NT * MAIN
ROWB = 1024
NBLK = NPAD // ROWB
TILE_ROWS = NPAD // NS

_mesh = plsc.VectorSubcoreMesh(core_axis_name="c", subcore_axis_name="s")
_sc_params = pltpu.CompilerParams(needs_layout_passes=False)


def _fill_flat_idx(edge_hbm, row, wid, buf):
    tbase = wid * MAIN
    pltpu.sync_copy(edge_hbm.at[row, pl.ds(tbase, MAIN)],
                    buf.at[pl.ds(0, MAIN)])
    pltpu.sync_copy(edge_hbm.at[row, pl.ds(EMAIN + wid * TAIL, TAIL)],
                    buf.at[pl.ds(MAIN, TAIL)])
    lanes = lax.iota(jnp.int32, 16)
    for i in range((NPAD - N) // 16):
        buf[pl.ds(N + 16 * i, 16)] = N + 16 * i + lanes


@functools.partial(
    pl.kernel,
    out_type=jax.ShapeDtypeStruct((NT, NPAD), jnp.float32),
    mesh=_mesh,
    compiler_params=_sc_params,
    scratch_types=[
        pltpu.VMEM((NPAD,), jnp.float32),
        pltpu.VMEM((NPAD,), jnp.int32),
    ],
)
def _sc_degree(edge_hbm, out_hbm, hist, dall):
    cid = lax.axis_index("c")
    sid = lax.axis_index("s")
    wid = cid * NS + sid
    zeros16 = jnp.zeros((16,), jnp.float32)
    ones16 = jnp.ones((16,), jnp.float32)
    _fill_flat_idx(edge_hbm, 1, wid, dall)

    def zero_body(i, _):
        for u in range(4):
            hist[pl.ds(i * 64 + u * 16, 16)] = zeros16
        return 0

    lax.fori_loop(0, NPAD // 64, zero_body, 0)

    def vec_body(i, _):
        for u in range(4):
            idx = dall[pl.ds(i * 64 + u * 16, 16)]
            plsc.addupdate_scatter(hist, [idx], ones16)
        return 0

    lax.fori_loop(0, NPAD // 64, vec_body, 0)
    pltpu.sync_copy(hist, out_hbm.at[wid])


@functools.partial(
    pl.kernel,
    out_type=jax.ShapeDtypeStruct((NC, NPAD, D_IN), jnp.float32),
    mesh=_mesh,
    compiler_params=_sc_params,
    scratch_types=[
        pltpu.VMEM_SHARED((NPAD, D_IN), jnp.float32),
        pltpu.VMEM((CHUNK, D_IN), jnp.float32),
        pltpu.VMEM((CHUNK, D_IN), jnp.float32),
        pltpu.VMEM((4, CHUNK), jnp.int32),
        pltpu.VMEM((K, CHUNK), jnp.int32),
        pltpu.VMEM((2, CHUNK), jnp.int32),
        pltpu.SemaphoreType.DMA,
        pltpu.SemaphoreType.DMA,
        pltpu.SemaphoreType.DMA,
        pltpu.SemaphoreType.DMA,
        pltpu.SemaphoreType.DMA,
        pltpu.SemaphoreType.DMA,
        pltpu.SemaphoreType.DMA,
        pltpu.SemaphoreType.DMA,
        pltpu.SemaphoreType.DMA,
    ],
)
def _sc_seg128(edge_hbm, a_hbm, out_hbm, acc, rows0, rows1, sring, dall,
               stail, gs0, gs1, ss0, ss1, is0, is1, is2, is3, ds0):
    cid = lax.axis_index("c")
    sid = lax.axis_index("s")
    wid = cid * NS + sid
    tbase = wid * MAIN
    zeros16 = jnp.zeros((16,), jnp.float32)
    lanes = lax.iota(jnp.int32, 16)

    def dfill_start(j, _):
        pltpu.async_copy(edge_hbm.at[1, pl.ds(tbase + j * CHUNK, CHUNK)],
                         dall.at[j], ds0)
        return 0

    def dfill_wait(j, _):
        pltpu.make_async_copy(edge_hbm.at[1, pl.ds(tbase + j * CHUNK, CHUNK)],
                              dall.at[j], ds0).wait()
        return 0

    lax.fori_loop(0, KM, dfill_start, 0)
    pltpu.sync_copy(edge_hbm.at[1, pl.ds(EMAIN + wid * TAIL, TAIL)],
                    dall.at[KM, pl.ds(0, TAIL)])
    pltpu.sync_copy(edge_hbm.at[0, pl.ds(EMAIN + wid * TAIL, TAIL)],
                    stail.at[0, pl.ds(0, TAIL)])
    for i in range(7):
        dall[KM, pl.ds(TAIL + 16 * i, 16)] = N + 16 * i + lanes
        stail[0, pl.ds(TAIL + 16 * i, 16)] = N + 16 * i + lanes
    for i in range(8):
        dall[KM + 1, pl.ds(16 * i, 16)] = N + 112 + 16 * i + lanes
        stail[1, pl.ds(16 * i, 16)] = N + 112 + 16 * i + lanes
    lax.fori_loop(0, KM, dfill_wait, 0)

    def zero_rows(r, _):
        for col in range(D_IN // 16):
            rows0[r, pl.ds(col * 16, 16)] = zeros16
        return 0

    lax.fori_loop(0, CHUNK, zero_rows, 0)
    base = sid * TILE_ROWS
    for k in range(TILE_ROWS // CHUNK):
        pltpu.sync_copy(rows0, acc.at[pl.ds(base + k * CHUNK, CHUNK)])
    plsc.subcore_barrier()

    isems = [is0, is1, is2, is3]

    def idx_start(jj, slot):
        pltpu.async_copy(edge_hbm.at[0, pl.ds(tbase + jj * CHUNK, CHUNK)],
                         sring.at[slot], isems[slot])

    def idx_wait(jj, slot):
        pltpu.make_async_copy(
            edge_hbm.at[0, pl.ds(tbase + jj * CHUNK, CHUNK)],
            sring.at[slot], isems[slot]).wait()

    def gat_start(slot, rows, gsem):
        pltpu.async_copy(a_hbm.at[sring.at[slot]], rows, gsem)

    def gat_wait(slot, rows, gsem):
        pltpu.make_async_copy(a_hbm.at[sring.at[slot]], rows, gsem).wait()

    def sca_start(jj, rows, ssem):
        pltpu.async_copy(rows, acc.at[dall.at[jj]], ssem, add=True)

    def sca_wait(jj, rows, ssem):
        pltpu.make_async_copy(rows, acc.at[dall.at[jj]], ssem).wait()

    pltpu.sync_copy(edge_hbm.at[0, pl.ds(tbase, CHUNK)], sring.at[0])
    pltpu.sync_copy(edge_hbm.at[0, pl.ds(tbase + CHUNK, CHUNK)], sring.at[1])
    gat_start(0, rows0, gs0)
    gat_start(1, rows1, gs1)
    idx_start(2, 2)
    idx_start(3, 3)

    def pipe_body(it, _):
        j = 4 * it
        gat_wait(0, rows0, gs0)
        idx_start(j + 4, 0)
        sca_start(j, rows0, ss0)
        gat_wait(1, rows1, gs1)
        idx_start(j + 5, 1)
        sca_start(j + 1, rows1, ss1)
        sca_wait(j, rows0, ss0)
        idx_wait(j + 2, 2)
        gat_start(2, rows0, gs0)
        sca_wait(j + 1, rows1, ss1)
        idx_wait(j + 3, 3)
        gat_start(3, rows1, gs1)
        gat_wait(2, rows0, gs0)
        idx_start(j + 6, 2)
        sca_start(j + 2, rows0, ss0)
        gat_wait(3, rows1, gs1)
        idx_start(j + 7, 3)
        sca_start(j + 3, rows1, ss1)
        sca_wait(j + 2, rows0, ss0)
        idx_wait(j + 4, 0)
        gat_start(0, rows0, gs0)
        sca_wait(j + 3, rows1, ss1)
        idx_wait(j + 5, 1)
        gat_start(1, rows1, gs1)
        return 0

    lax.fori_loop(0, (K - 4) // 4 - 1, pipe_body, 0)

    j = K - 8
    gat_wait(0, rows0, gs0)
    idx_start(j + 4, 0)
    sca_start(j, rows0, ss0)
    gat_wait(1, rows1, gs1)
    idx_start(j + 5, 1)
    sca_start(j + 1, rows1, ss1)
    sca_wait(j, rows0, ss0)
    idx_wait(j + 2, 2)
    gat_start(2, rows0, gs0)
    sca_wait(j + 1, rows1, ss1)
    idx_wait(j + 3, 3)
    gat_start(3, rows1, gs1)
    gat_wait(2, rows0, gs0)
    sca_start(j + 2, rows0, ss0)
    gat_wait(3, rows1, gs1)
    sca_start(j + 3, rows1, ss1)
    sca_wait(j + 2, rows0, ss0)
    idx_wait(j + 4, 0)
    gat_start(0, rows0, gs0)
    sca_wait(j + 3, rows1, ss1)
    idx_wait(j + 5, 1)
    gat_start(1, rows1, gs1)

    jl = K - 4
    gat_wait(0, rows0, gs0)
    sca_start(jl, rows0, ss0)
    gat_wait(1, rows1, gs1)
    sca_start(jl + 1, rows1, ss1)
    sca_wait(jl, rows0, ss0)
    pltpu.async_copy(a_hbm.at[stail.at[0]], rows0, gs0)
    sca_wait(jl + 1, rows1, ss1)
    pltpu.async_copy(a_hbm.at[stail.at[1]], rows1, gs1)
    pltpu.make_async_copy(a_hbm.at[stail.at[0]], rows0, gs0).wait()
    sca_start(jl + 2, rows0, ss0)
    pltpu.make_async_copy(a_hbm.at[stail.at[1]], rows1, gs1).wait()
    sca_start(jl + 3, rows1, ss1)
    sca_wait(jl + 2, rows0, ss0)
    sca_wait(jl + 3, rows1, ss1)

    plsc.subcore_barrier()
    pltpu.sync_copy(acc.at[pl.ds(base, TILE_ROWS)],
                    out_hbm.at[cid, pl.ds(base, TILE_ROWS)])


@functools.partial(
    pl.kernel,
    out_type=jax.ShapeDtypeStruct((NT, NPAD), jnp.float32),
    mesh=_mesh,
    compiler_params=_sc_params,
    scratch_types=[
        pltpu.VMEM((NBLK, ROWB // CHUNK, CHUNK), jnp.float32),
        pltpu.VMEM((NPAD,), jnp.float32),
        pltpu.VMEM((NPAD,), jnp.int32),
        pltpu.VMEM((NPAD,), jnp.int32),
    ],
)
def _sc_segscalar(edge_hbm, c_hbm, out_hbm, cloc, tloc, sall, dall):
    cid = lax.axis_index("c")
    sid = lax.axis_index("s")
    wid = cid * NS + sid
    zeros16 = jnp.zeros((16,), jnp.float32)
    _fill_flat_idx(edge_hbm, 0, wid, sall)
    _fill_flat_idx(edge_hbm, 1, wid, dall)
    pltpu.sync_copy(c_hbm, cloc)

    def zero_body(i, _):
        for u in range(4):
            tloc[pl.ds(i * 64 + u * 16, 16)] = zeros16
        return 0

    lax.fori_loop(0, NPAD // 64, zero_body, 0)

    def vec_body(i, _):
        for u in range(4):
            sidx = sall[pl.ds(i * 64 + u * 16, 16)]
            didx = dall[pl.ds(i * 64 + u * 16, 16)]
            v = plsc.load_gather(
                cloc, [lax.shift_right_logical(sidx, 9),
                       lax.bitwise_and(lax.shift_right_logical(sidx, 7), 3),
                       lax.bitwise_and(sidx, 127)])
            plsc.addupdate_scatter(tloc, [didx], v)
        return 0

    lax.fori_loop(0, NPAD // 64, vec_body, 0)
    pltpu.sync_copy(tloc, out_hbm.at[wid])


def _tc_prep_body(uy_ref, x_ref, degp_ref, a_ref, dinv_ref):
    deg = 1.0 + jnp.sum(degp_ref[...], axis=0)
    dinv = lax.rsqrt(deg)[:, None]
    latent = jnp.concatenate([uy_ref[...], x_ref[...]], axis=1)
    latent = jnp.concatenate(
        [latent, jnp.zeros((NPAD - N, D_IN), jnp.float32)], axis=0)
    a_ref[...] = latent * dinv
    dinv_ref[...] = dinv


def _tc_prep(u_Y, X, deg_parts):
    return pl.pallas_call(
        _tc_prep_body,
        out_shape=[
            jax.ShapeDtypeStruct((NPAD, D_IN), jnp.float32),
            jax.ShapeDtypeStruct((NPAD, 1), jnp.float32),
        ],
    )(u_Y, X, deg_parts)


def _tc_mlp_body(parts_ref, a_ref, dinv_ref, w1_ref, b1_ref, w2_ref,
                 c80_ref):
    i = pl.program_id(0)
    dinv = dinv_ref[...]
    z = (parts_ref[0] + parts_ref[1] + a_ref[...]) * dinv
    h = jnp.maximum(
        jnp.dot(z, w1_ref[...], preferred_element_type=jnp.float32)
        + b1_ref[...],
        0.0,
    )
    w2d = w2_ref[:, 0:1] - w2_ref[:, 1:2]
    c = jnp.dot(h, w2d, preferred_element_type=jnp.float32) * dinv
    row = i * ROWB + lax.broadcasted_iota(jnp.int32, (ROWB, 1), 0)
    c = jnp.where(row < N, c, 0.0)
    c80_ref[...] = c.reshape(1, ROWB // CHUNK, CHUNK)


def _tc_mlp(parts, a, dinv, w1, b1, w2):
    return pl.pallas_call(
        _tc_mlp_body,
        grid=(NBLK,),
        in_specs=[
            pl.BlockSpec((NC, ROWB, D_IN), lambda i: (0, i, 0)),
            pl.BlockSpec((ROWB, D_IN), lambda i: (i, 0)),
            pl.BlockSpec((ROWB, 1), lambda i: (i, 0)),
            pl.BlockSpec((D_IN, HID), lambda i: (0, 0)),
            pl.BlockSpec((1, HID), lambda i: (0, 0)),
            pl.BlockSpec((HID, 2), lambda i: (0, 0)),
        ],
        out_specs=pl.BlockSpec((1, ROWB // CHUNK, CHUNK),
                               lambda i: (i, 0, 0)),
        out_shape=jax.ShapeDtypeStruct((NBLK, ROWB // CHUNK, CHUNK),
                                       jnp.float32),
    )(parts, a, dinv, w1, b1, w2)


def _tc_finish_body(tp_ref, c80_ref, dinv_ref, b2_ref, out_ref):
    t = jnp.sum(tp_ref[...], axis=0).reshape(ROWB // CHUNK, CHUNK)
    dinv = dinv_ref[...].reshape(ROWB // CHUNK, CHUNK)
    delta = dinv * (t + c80_ref[0]) + (b2_ref[0, 0] - b2_ref[0, 1])
    pos = delta >= 0.0
    ez = jnp.exp(jnp.where(pos, -delta, delta))
    p0 = jnp.where(pos, 1.0 / (1.0 + ez), ez / (1.0 + ez))
    out_ref[...] = jnp.concatenate(
        [p0.reshape(ROWB, 1), (1.0 - p0).reshape(ROWB, 1)], axis=1)


def _tc_finish(t_parts, c80, dinv, b2):
    return pl.pallas_call(
        _tc_finish_body,
        grid=(NBLK,),
        in_specs=[
            pl.BlockSpec((NT, ROWB), lambda i: (0, i)),
            pl.BlockSpec((1, ROWB // CHUNK, CHUNK), lambda i: (i, 0, 0)),
            pl.BlockSpec((ROWB, 1), lambda i: (i, 0)),
            pl.BlockSpec((1, 2), lambda i: (0, 0)),
        ],
        out_specs=pl.BlockSpec((ROWB, 2), lambda i: (i, 0)),
        out_shape=jax.ShapeDtypeStruct((NPAD, 2), jnp.float32),
    )(t_parts, c80, dinv, b2)


@jax.jit
def kernel(edge_index, X, u_Y, W1, b1, W2, b2):
    deg_parts = _sc_degree(edge_index)
    a, dinv = _tc_prep(u_Y, X, deg_parts)
    parts = _sc_seg128(edge_index, a)
    c80 = _tc_mlp(parts, a, dinv, W1, b1.reshape(1, HID), W2)
    t_parts = _sc_segscalar(edge_index, c80)
    out = _tc_finish(t_parts, c80, dinv, b2.reshape(1, 2))
    return out[:N]


# speedup vs baseline: 1.1681x; 1.0066x over previous
import functools

import jax
import jax.numpy as jnp
from jax import lax
from jax.experimental import pallas as pl
from jax.experimental.pallas import tpu as pltpu
from jax.experimental.pallas import tpu_sc as plsc

N = 10000
E = 320000
D_IN = 128
HID = 512
NPAD = 10240
NC = 2
NS = 16
NT = NC * NS
CHUNK = 128
K = 80
KM = 78
MAIN = KM * CHUNK
TAIL = 16
EMAIN = NT * MAIN
ROWB = 1024
NBLK = NPAD // ROWB
TILE_ROWS = NPAD // NS

_mesh = plsc.VectorSubcoreMesh(core_axis_name="c", subcore_axis_name="s")
_sc_params = pltpu.CompilerParams(needs_layout_passes=False)


def _fill_flat_idx(edge_hbm, row, wid, buf):
    tbase = wid * MAIN
    pltpu.sync_copy(edge_hbm.at[row, pl.ds(tbase, MAIN)],
                    buf.at[pl.ds(0, MAIN)])
    pltpu.sync_copy(edge_hbm.at[row, pl.ds(EMAIN + wid * TAIL, TAIL)],
                    buf.at[pl.ds(MAIN, TAIL)])
    lanes = lax.iota(jnp.int32, 16)
    for i in range((NPAD - N) // 16):
        buf[pl.ds(N + 16 * i, 16)] = N + 16 * i + lanes


@functools.partial(
    pl.kernel,
    out_type=jax.ShapeDtypeStruct((NT, NPAD), jnp.float32),
    mesh=_mesh,
    compiler_params=_sc_params,
    scratch_types=[
        pltpu.VMEM((NPAD,), jnp.float32),
        pltpu.VMEM((NPAD,), jnp.int32),
    ],
)
def _sc_degree(edge_hbm, out_hbm, hist, dall):
    cid = lax.axis_index("c")
    sid = lax.axis_index("s")
    wid = cid * NS + sid
    zeros16 = jnp.zeros((16,), jnp.float32)
    ones16 = jnp.ones((16,), jnp.float32)
    _fill_flat_idx(edge_hbm, 1, wid, dall)

    def zero_body(i, _):
        for u in range(4):
            hist[pl.ds(i * 64 + u * 16, 16)] = zeros16
        return 0

    lax.fori_loop(0, NPAD // 64, zero_body, 0)

    def vec_body(i, _):
        for u in range(4):
            idx = dall[pl.ds(i * 64 + u * 16, 16)]
            plsc.addupdate_scatter(hist, [idx], ones16)
        return 0

    lax.fori_loop(0, NPAD // 64, vec_body, 0)
    pltpu.sync_copy(hist, out_hbm.at[wid])


@functools.partial(
    pl.kernel,
    out_type=jax.ShapeDtypeStruct((NC, NPAD, D_IN), jnp.float32),
    mesh=_mesh,
    compiler_params=_sc_params,
    scratch_types=[
        pltpu.VMEM_SHARED((NPAD, D_IN), jnp.float32),
        pltpu.VMEM((CHUNK, D_IN), jnp.float32),
        pltpu.VMEM((CHUNK, D_IN), jnp.float32),
        pltpu.VMEM((4, CHUNK), jnp.int32),
        pltpu.VMEM((K, CHUNK), jnp.int32),
        pltpu.VMEM((2, CHUNK), jnp.int32),
        pltpu.SemaphoreType.DMA,
        pltpu.SemaphoreType.DMA,
        pltpu.SemaphoreType.DMA,
        pltpu.SemaphoreType.DMA,
        pltpu.SemaphoreType.DMA,
        pltpu.SemaphoreType.DMA,
        pltpu.SemaphoreType.DMA,
        pltpu.SemaphoreType.DMA,
        pltpu.SemaphoreType.DMA,
    ],
)
def _sc_seg128(edge_hbm, a_hbm, out_hbm, acc, rows0, rows1, sring, dall,
               stail, gs0, gs1, ss0, ss1, is0, is1, is2, is3, ds0):
    cid = lax.axis_index("c")
    sid = lax.axis_index("s")
    wid = cid * NS + sid
    tbase = wid * MAIN
    zeros16 = jnp.zeros((16,), jnp.float32)
    lanes = lax.iota(jnp.int32, 16)

    def dfill_start(j, _):
        pltpu.async_copy(edge_hbm.at[1, pl.ds(tbase + j * CHUNK, CHUNK)],
                         dall.at[j], ds0)
        return 0

    def dfill_wait(j, _):
        pltpu.make_async_copy(edge_hbm.at[1, pl.ds(tbase + j * CHUNK, CHUNK)],
                              dall.at[j], ds0).wait()
        return 0

    lax.fori_loop(0, KM, dfill_start, 0)
    pltpu.sync_copy(edge_hbm.at[1, pl.ds(EMAIN + wid * TAIL, TAIL)],
                    dall.at[KM, pl.ds(0, TAIL)])
    pltpu.sync_copy(edge_hbm.at[0, pl.ds(EMAIN + wid * TAIL, TAIL)],
                    stail.at[0, pl.ds(0, TAIL)])
    for i in range(7):
        dall[KM, pl.ds(TAIL + 16 * i, 16)] = N + 16 * i + lanes
        stail[0, pl.ds(TAIL + 16 * i, 16)] = N + 16 * i + lanes
    for i in range(8):
        dall[KM + 1, pl.ds(16 * i, 16)] = N + 112 + 16 * i + lanes
        stail[1, pl.ds(16 * i, 16)] = N + 112 + 16 * i + lanes
    lax.fori_loop(0, KM, dfill_wait, 0)

    def zero_rows(r, _):
        for col in range(D_IN // 16):
            rows0[r, pl.ds(col * 16, 16)] = zeros16
        return 0

    lax.fori_loop(0, CHUNK, zero_rows, 0)
    base = sid * TILE_ROWS
    for k in range(TILE_ROWS // CHUNK):
        pltpu.sync_copy(rows0, acc.at[pl.ds(base + k * CHUNK, CHUNK)])
    plsc.subcore_barrier()

    isems = [is0, is1, is2, is3]

    def idx_start(jj, slot):
        pltpu.async_copy(edge_hbm.at[0, pl.ds(tbase + jj * CHUNK, CHUNK)],
                         sring.at[slot], isems[slot])

    def idx_wait(jj, slot):
        pltpu.make_async_copy(
            edge_hbm.at[0, pl.ds(tbase + jj * CHUNK, CHUNK)],
            sring.at[slot], isems[slot]).wait()

    def gat_start(slot, rows, gsem):
        pltpu.async_copy(a_hbm.at[sring.at[slot]], rows, gsem)

    def gat_wait(slot, rows, gsem):
        pltpu.make_async_copy(a_hbm.at[sring.at[slot]], rows, gsem).wait()

    def sca_start(jj, rows, ssem):
        pltpu.async_copy(rows, acc.at[dall.at[jj]], ssem, add=True)

    def sca_wait(jj, rows, ssem):
        pltpu.make_async_copy(rows, acc.at[dall.at[jj]], ssem).wait()

    pltpu.sync_copy(edge_hbm.at[0, pl.ds(tbase, CHUNK)], sring.at[0])
    pltpu.sync_copy(edge_hbm.at[0, pl.ds(tbase + CHUNK, CHUNK)], sring.at[1])
    gat_start(0, rows0, gs0)
    gat_start(1, rows1, gs1)
    idx_start(2, 2)
    idx_start(3, 3)

    def pipe_body(it, _):
        j = 4 * it
        gat_wait(0, rows0, gs0)
        idx_start(j + 4, 0)
        sca_start(j, rows0, ss0)
        gat_wait(1, rows1, gs1)
        idx_start(j + 5, 1)
        sca_start(j + 1, rows1, ss1)
        sca_wait(j, rows0, ss0)
        idx_wait(j + 2, 2)
        gat_start(2, rows0, gs0)
        sca_wait(j + 1, rows1, ss1)
        idx_wait(j + 3, 3)
        gat_start(3, rows1, gs1)
        gat_wait(2, rows0, gs0)
        idx_start(j + 6, 2)
        sca_start(j + 2, rows0, ss0)
        gat_wait(3, rows1, gs1)
        idx_start(j + 7, 3)
        sca_start(j + 3, rows1, ss1)
        sca_wait(j + 2, rows0, ss0)
        idx_wait(j + 4, 0)
        gat_start(0, rows0, gs0)
        sca_wait(j + 3, rows1, ss1)
        idx_wait(j + 5, 1)
        gat_start(1, rows1, gs1)
        return 0

    lax.fori_loop(0, (K - 4) // 4 - 1, pipe_body, 0)

    j = K - 8
    gat_wait(0, rows0, gs0)
    idx_start(j + 4, 0)
    sca_start(j, rows0, ss0)
    gat_wait(1, rows1, gs1)
    idx_start(j + 5, 1)
    sca_start(j + 1, rows1, ss1)
    sca_wait(j, rows0, ss0)
    idx_wait(j + 2, 2)
    gat_start(2, rows0, gs0)
    sca_wait(j + 1, rows1, ss1)
    idx_wait(j + 3, 3)
    gat_start(3, rows1, gs1)
    gat_wait(2, rows0, gs0)
    sca_start(j + 2, rows0, ss0)
    gat_wait(3, rows1, gs1)
    sca_start(j + 3, rows1, ss1)
    sca_wait(j + 2, rows0, ss0)
    idx_wait(j + 4, 0)
    gat_start(0, rows0, gs0)
    sca_wait(j + 3, rows1, ss1)
    idx_wait(j + 5, 1)
    gat_start(1, rows1, gs1)

    jl = K - 4
    gat_wait(0, rows0, gs0)
    sca_start(jl, rows0, ss0)
    gat_wait(1, rows1, gs1)
    sca_start(jl + 1, rows1, ss1)
    sca_wait(jl, rows0, ss0)
    pltpu.async_copy(a_hbm.at[stail.at[0]], rows0, gs0)
    sca_wait(jl + 1, rows1, ss1)
    pltpu.async_copy(a_hbm.at[stail.at[1]], rows1, gs1)
    pltpu.make_async_copy(a_hbm.at[stail.at[0]], rows0, gs0).wait()
    sca_start(jl + 2, rows0, ss0)
    pltpu.make_async_copy(a_hbm.at[stail.at[1]], rows1, gs1).wait()
    sca_start(jl + 3, rows1, ss1)
    sca_wait(jl + 2, rows0, ss0)
    sca_wait(jl + 3, rows1, ss1)

    plsc.subcore_barrier()
    pltpu.sync_copy(acc.at[pl.ds(base, TILE_ROWS)],
                    out_hbm.at[cid, pl.ds(base, TILE_ROWS)])


@functools.partial(
    pl.kernel,
    out_type=jax.ShapeDtypeStruct((NT, NPAD), jnp.float32),
    mesh=_mesh,
    compiler_params=_sc_params,
    scratch_types=[
        pltpu.VMEM((NBLK, ROWB // CHUNK, CHUNK), jnp.float32),
        pltpu.VMEM((NPAD,), jnp.float32),
        pltpu.VMEM((NPAD,), jnp.int32),
        pltpu.VMEM((NPAD,), jnp.int32),
    ],
)
def _sc_segscalar(edge_hbm, c_hbm, out_hbm, cloc, tloc, sall, dall):
    cid = lax.axis_index("c")
    sid = lax.axis_index("s")
    wid = cid * NS + sid
    zeros16 = jnp.zeros((16,), jnp.float32)
    _fill_flat_idx(edge_hbm, 0, wid, sall)
    _fill_flat_idx(edge_hbm, 1, wid, dall)
    pltpu.sync_copy(c_hbm, cloc)

    def zero_body(i, _):
        for u in range(4):
            tloc[pl.ds(i * 64 + u * 16, 16)] = zeros16
        return 0

    lax.fori_loop(0, NPAD // 64, zero_body, 0)

    def vec_body(i, _):
        for u in range(4):
            sidx = sall[pl.ds(i * 64 + u * 16, 16)]
            didx = dall[pl.ds(i * 64 + u * 16, 16)]
            v = plsc.load_gather(
                cloc, [lax.shift_right_logical(sidx, 10),
                       lax.bitwise_and(lax.shift_right_logical(sidx, 7), 7),
                       lax.bitwise_and(sidx, 127)])
            plsc.addupdate_scatter(tloc, [didx], v)
        return 0

    lax.fori_loop(0, NPAD // 64, vec_body, 0)
    pltpu.sync_copy(tloc, out_hbm.at[wid])


def _tc_prep_body(uy_ref, x_ref, degp_ref, a_ref, dinv_ref):
    deg = 1.0 + jnp.sum(degp_ref[...], axis=0)
    dinv = lax.rsqrt(deg)[:, None]
    latent = jnp.concatenate([uy_ref[...], x_ref[...]], axis=1)
    latent = jnp.concatenate(
        [latent, jnp.zeros((NPAD - N, D_IN), jnp.float32)], axis=0)
    a_ref[...] = latent * dinv
    dinv_ref[...] = dinv


def _tc_prep(u_Y, X, deg_parts):
    return pl.pallas_call(
        _tc_prep_body,
        out_shape=[
            jax.ShapeDtypeStruct((NPAD, D_IN), jnp.float32),
            jax.ShapeDtypeStruct((NPAD, 1), jnp.float32),
        ],
    )(u_Y, X, deg_parts)


def _tc_mlp_body(parts_ref, a_ref, dinv_ref, w1_ref, b1_ref, w2_ref,
                 c80_ref):
    i = pl.program_id(0)
    dinv = dinv_ref[...]
    z = (parts_ref[0] + parts_ref[1] + a_ref[...]) * dinv
    h = jnp.maximum(
        jnp.dot(z, w1_ref[...], preferred_element_type=jnp.float32)
        + b1_ref[...],
        0.0,
    )
    w2d = w2_ref[:, 0:1] - w2_ref[:, 1:2]
    c = jnp.dot(h, w2d, preferred_element_type=jnp.float32) * dinv
    row = i * ROWB + lax.broadcasted_iota(jnp.int32, (ROWB, 1), 0)
    c = jnp.where(row < N, c, 0.0)
    c80_ref[...] = c.reshape(1, ROWB // CHUNK, CHUNK)


def _tc_mlp(parts, a, dinv, w1, b1, w2):
    return pl.pallas_call(
        _tc_mlp_body,
        grid=(NBLK,),
        in_specs=[
            pl.BlockSpec((NC, ROWB, D_IN), lambda i: (0, i, 0)),
            pl.BlockSpec((ROWB, D_IN), lambda i: (i, 0)),
            pl.BlockSpec((ROWB, 1), lambda i: (i, 0)),
            pl.BlockSpec((D_IN, HID), lambda i: (0, 0)),
            pl.BlockSpec((1, HID), lambda i: (0, 0)),
            pl.BlockSpec((HID, 2), lambda i: (0, 0)),
        ],
        out_specs=pl.BlockSpec((1, ROWB // CHUNK, CHUNK),
                               lambda i: (i, 0, 0)),
        out_shape=jax.ShapeDtypeStruct((NBLK, ROWB // CHUNK, CHUNK),
                                       jnp.float32),
    )(parts, a, dinv, w1, b1, w2)


def _tc_finish_body(tp_ref, c80_ref, dinv_ref, b2_ref, out_ref):
    t = jnp.sum(tp_ref[...], axis=0).reshape(ROWB // CHUNK, CHUNK)
    dinv = dinv_ref[...].reshape(ROWB // CHUNK, CHUNK)
    delta = dinv * (t + c80_ref[0]) + (b2_ref[0, 0] - b2_ref[0, 1])
    pos = delta >= 0.0
    ez = jnp.exp(jnp.where(pos, -delta, delta))
    p0 = jnp.where(pos, 1.0 / (1.0 + ez), ez / (1.0 + ez))
    out_ref[...] = jnp.concatenate(
        [p0.reshape(ROWB, 1), (1.0 - p0).reshape(ROWB, 1)], axis=1)


def _tc_finish(t_parts, c80, dinv, b2):
    return pl.pallas_call(
        _tc_finish_body,
        grid=(NBLK,),
        in_specs=[
            pl.BlockSpec((NT, ROWB), lambda i: (0, i)),
            pl.BlockSpec((1, ROWB // CHUNK, CHUNK), lambda i: (i, 0, 0)),
            pl.BlockSpec((ROWB, 1), lambda i: (i, 0)),
            pl.BlockSpec((1, 2), lambda i: (0, 0)),
        ],
        out_specs=pl.BlockSpec((ROWB, 2), lambda i: (i, 0)),
        out_shape=jax.ShapeDtypeStruct((NPAD, 2), jnp.float32),
    )(t_parts, c80, dinv, b2)


@jax.jit
def kernel(edge_index, X, u_Y, W1, b1, W2, b2):
    deg_parts = _sc_degree(edge_index)
    a, dinv = _tc_prep(u_Y, X, deg_parts)
    parts = _sc_seg128(edge_index, a)
    c80 = _tc_mlp(parts, a, dinv, W1, b1.reshape(1, HID), W2)
    t_parts = _sc_segscalar(edge_index, c80)
    out = _tc_finish(t_parts, c80, dinv, b2.reshape(1, 2))
    return out[:N]
